# baseline (speedup 1.0000x reference)
import functools

import jax
import jax.numpy as jnp
from jax import lax
from jax.experimental import pallas as pl
from jax.experimental.pallas import tpu as pltpu
from jax.experimental.pallas import tpu_sc as plsc

N_ATOMS = 100000
D_FEAT = 256
D_HIDDEN = 256
N_MOLS = 4096

NW = 32
L = 16
PAD_ATOMS = 102400
CHUNK = PAD_ATOMS // NW
NV = CHUNK // L
ACC = N_MOLS + L
MB = N_MOLS // NW

ATOM_BLOCK = 10240


def _mlp_body(vi_ref, w1_ref, b1_ref, w2_ref, b2_ref, out_ref):
    x = vi_ref[...]
    h = jnp.dot(x, w1_ref[...], preferred_element_type=jnp.float32)
    h = jnp.maximum(h + b1_ref[...], 0.0)
    p = jax.lax.dot_general(
        w2_ref[...],
        h,
        dimension_numbers=(((1,), (1,)), ((), ())),
        preferred_element_type=jnp.float32,
    )
    out_ref[...] = p[0, :] + b2_ref[0, 0]


def _mlp(vi, W1, b1, W2, b2):
    grid = (PAD_ATOMS // ATOM_BLOCK,)
    return pl.pallas_call(
        _mlp_body,
        grid=grid,
        in_specs=[
            pl.BlockSpec((ATOM_BLOCK, D_FEAT), lambda i: (i, 0)),
            pl.BlockSpec((D_FEAT, D_HIDDEN), lambda i: (0, 0)),
            pl.BlockSpec((1, D_HIDDEN), lambda i: (0, 0)),
            pl.BlockSpec((1, D_HIDDEN), lambda i: (0, 0)),
            pl.BlockSpec((1, 1), lambda i: (0, 0)),
        ],
        out_specs=pl.BlockSpec((ATOM_BLOCK,), lambda i: (i,)),
        out_shape=jax.ShapeDtypeStruct((PAD_ATOMS,), jnp.float32),
        compiler_params=pltpu.CompilerParams(vmem_limit_bytes=128 * 1024 * 1024),
    )(vi, W1, b1.reshape(1, D_HIDDEN), W2.reshape(1, D_HIDDEN), b2.reshape(1, 1))


def _sc_mesh():
    return plsc.VectorSubcoreMesh(
        core_axis_name="c", subcore_axis_name="s", num_cores=2, num_subcores=16
    )


def _pool_partial(idx_pad, val_pad):
    @functools.partial(
        pl.kernel,
        out_type=(
            jax.ShapeDtypeStruct((NW, N_MOLS), jnp.float32),
            jax.ShapeDtypeStruct((NW, N_MOLS), jnp.float32),
        ),
        mesh=_sc_mesh(),
        compiler_params=pltpu.CompilerParams(needs_layout_passes=False),
        scratch_types=[
            pltpu.VMEM((CHUNK,), jnp.int32),
            pltpu.VMEM((CHUNK,), jnp.float32),
            pltpu.VMEM((ACC,), jnp.float32),
            pltpu.VMEM((ACC,), jnp.float32),
            pltpu.VMEM((ACC,), jnp.float32),
            pltpu.VMEM((ACC,), jnp.float32),
        ],
    )
    def body(idx_hbm, val_hbm, psum_hbm, pcnt_hbm, idx_v, val_v, se, ss, pe, ps):
        wid = lax.axis_index("s") * 2 + lax.axis_index("c")
        base = wid * CHUNK
        pltpu.sync_copy(idx_hbm.at[pl.ds(base, CHUNK)], idx_v)
        pltpu.sync_copy(val_hbm.at[pl.ds(base, CHUNK)], val_v)

        zero = jnp.zeros((L,), jnp.float32)

        def zbody(i, _):
            sl = pl.ds(i * L, L)
            se[sl] = zero
            ss[sl] = zero
            pe[sl] = zero
            ps[sl] = zero
            return 0

        lax.fori_loop(0, ACC // L, zbody, 0)

        lanes = lax.iota(jnp.int32, L)
        perm_next = jnp.minimum(lanes + 1, L - 1)
        perm_prev = jnp.maximum(lanes - 1, 0)

        def vbody(j, carry):
            carry_sum, prev_d = carry
            sl = pl.ds(j * L, L)
            d = idx_v[sl]
            v = val_v[sl]
            d_next = jnp.where(
                lanes == L - 1,
                -1,
                jnp.take_along_axis(d, perm_next, axis=0, mode="promise_in_bounds"),
            )
            d_prev = jnp.where(
                lanes == 0,
                prev_d,
                jnp.take_along_axis(d, perm_prev, axis=0, mode="promise_in_bounds"),
            )
            end_m = d != d_next
            start_m = d != d_prev
            c = plsc.cumsum(v) + carry_sum
            excl = c - v
            pos = (base + j * L + lanes).astype(jnp.float32)
            plsc.store_scatter(se, [d], c, mask=end_m)
            plsc.store_scatter(ss, [d], excl, mask=start_m)
            plsc.store_scatter(pe, [d], pos + 1.0, mask=end_m)
            plsc.store_scatter(ps, [d], pos, mask=start_m)
            return (carry_sum + jnp.sum(v), jnp.max(d))

        lax.fori_loop(0, NV, vbody, (jnp.float32(0.0), jnp.int32(-1)))

        def dbody(i, _):
            sl = pl.ds(i * L, L)
            se[sl] = se[sl] - ss[sl]
            pe[sl] = pe[sl] - ps[sl]
            return 0

        lax.fori_loop(0, N_MOLS // L, dbody, 0)

        pltpu.sync_copy(se.at[pl.ds(0, N_MOLS)], psum_hbm.at[wid])
        pltpu.sync_copy(pe.at[pl.ds(0, N_MOLS)], pcnt_hbm.at[wid])

    return body(idx_pad, val_pad)


def _pool_combine(psum, pcnt):
    @functools.partial(
        pl.kernel,
        out_type=jax.ShapeDtypeStruct((N_MOLS,), jnp.float32),
        mesh=_sc_mesh(),
        compiler_params=pltpu.CompilerParams(needs_layout_passes=False),
        scratch_types=[
            pltpu.VMEM((NW, MB), jnp.float32),
            pltpu.VMEM((NW, MB), jnp.float32),
            pltpu.VMEM((MB,), jnp.float32),
        ],
    )
    def body(psum_hbm, pcnt_hbm, out_hbm, s_v, c_v, o_v):
        wid = lax.axis_index("s") * 2 + lax.axis_index("c")
        col = wid * MB
        pltpu.sync_copy(psum_hbm.at[:, pl.ds(col, MB)], s_v)
        pltpu.sync_copy(pcnt_hbm.at[:, pl.ds(col, MB)], c_v)

        zero = jnp.zeros((L,), jnp.float32)

        def gbody(g, _):
            sl = pl.ds(g * L, L)

            def rbody(r, acc):
                sa, ca = acc
                return (sa + s_v[r, sl], ca + c_v[r, sl])

            s, c = lax.fori_loop(0, NW, rbody, (zero, zero))
            o_v[sl] = s / jnp.maximum(c, 1.0)
            return 0

        lax.fori_loop(0, MB // L, gbody, 0)
        pltpu.sync_copy(o_v, out_hbm.at[pl.ds(col, MB)])

    return body(psum, pcnt)


def kernel(vi, atom_mol_batch, W1, b1, W2, b2):
    val_pad = _mlp(vi, W1, b1, W2, b2)
    idx_pad = jnp.pad(
        atom_mol_batch.astype(jnp.int32),
        (0, PAD_ATOMS - N_ATOMS),
        constant_values=N_MOLS,
    )
    psum, pcnt = _pool_partial(idx_pad, val_pad)
    mol_prop = _pool_combine(psum, pcnt)
    return mol_prop[:, None]


# speedup vs baseline: 1.0375x; 1.0375x over previous
import functools

import jax
import jax.numpy as jnp
from jax import lax
from jax.experimental import pallas as pl
from jax.experimental.pallas import tpu as pltpu
from jax.experimental.pallas import tpu_sc as plsc

N_ATOMS = 100000
D_FEAT = 256
D_HIDDEN = 256
N_MOLS = 4096

NW = 32
L = 16
CHUNK = 3136
NV = CHUNK // L
TAIL = N_ATOMS - (NW - 1) * CHUNK
TAIL_NV = TAIL // L
ACC = N_MOLS
MB = N_MOLS // NW

ATOM_BLOCK = 14336
N_BLOCKS = -(-N_ATOMS // ATOM_BLOCK)


def _mlp_body(vi_ref, w1_ref, b1_ref, w2_ref, b2_ref, out_ref):
    x = vi_ref[...]
    h = jnp.dot(x, w1_ref[...], preferred_element_type=jnp.float32)
    h = jnp.maximum(h + b1_ref[...], 0.0)
    p = jax.lax.dot_general(
        w2_ref[...],
        h,
        dimension_numbers=(((1,), (1,)), ((), ())),
        preferred_element_type=jnp.float32,
    )
    out_ref[...] = p[0, :] + b2_ref[0, 0]


def _mlp(vi, W1, b1, W2, b2):
    return pl.pallas_call(
        _mlp_body,
        grid=(N_BLOCKS,),
        in_specs=[
            pl.BlockSpec((ATOM_BLOCK, D_FEAT), lambda i: (i, 0)),
            pl.BlockSpec((D_FEAT, D_HIDDEN), lambda i: (0, 0)),
            pl.BlockSpec((1, D_HIDDEN), lambda i: (0, 0)),
            pl.BlockSpec((1, D_HIDDEN), lambda i: (0, 0)),
            pl.BlockSpec((1, 1), lambda i: (0, 0)),
        ],
        out_specs=pl.BlockSpec((ATOM_BLOCK,), lambda i: (i,)),
        out_shape=jax.ShapeDtypeStruct((N_ATOMS,), jnp.float32),
        compiler_params=pltpu.CompilerParams(vmem_limit_bytes=128 * 1024 * 1024),
    )(vi, W1, b1.reshape(1, D_HIDDEN), W2.reshape(1, D_HIDDEN), b2.reshape(1, 1))


def _sc_mesh():
    return plsc.VectorSubcoreMesh(
        core_axis_name="c", subcore_axis_name="s", num_cores=2, num_subcores=16
    )


def _pool_partial(idx, val):
    @functools.partial(
        pl.kernel,
        out_type=(
            jax.ShapeDtypeStruct((NW, N_MOLS), jnp.float32),
            jax.ShapeDtypeStruct((NW, N_MOLS), jnp.float32),
        ),
        mesh=_sc_mesh(),
        compiler_params=pltpu.CompilerParams(needs_layout_passes=False),
        scratch_types=[
            pltpu.VMEM((CHUNK,), jnp.int32),
            pltpu.VMEM((CHUNK,), jnp.float32),
            pltpu.VMEM((ACC,), jnp.float32),
            pltpu.VMEM((ACC,), jnp.float32),
            pltpu.VMEM((ACC,), jnp.float32),
            pltpu.VMEM((ACC,), jnp.float32),
        ],
    )
    def body(idx_hbm, val_hbm, psum_hbm, pcnt_hbm, idx_v, val_v, se, ss, pe, ps):
        wid = lax.axis_index("s") * 2 + lax.axis_index("c")
        base = wid * CHUNK
        last = wid == NW - 1
        pltpu.sync_copy(idx_hbm.at[pl.ds(base, TAIL)], idx_v.at[pl.ds(0, TAIL)])
        pltpu.sync_copy(val_hbm.at[pl.ds(base, TAIL)], val_v.at[pl.ds(0, TAIL)])

        @pl.when(jnp.logical_not(last))
        def _():
            pltpu.sync_copy(
                idx_hbm.at[pl.ds(base + TAIL, CHUNK - TAIL)],
                idx_v.at[pl.ds(TAIL, CHUNK - TAIL)],
            )
            pltpu.sync_copy(
                val_hbm.at[pl.ds(base + TAIL, CHUNK - TAIL)],
                val_v.at[pl.ds(TAIL, CHUNK - TAIL)],
            )

        zero = jnp.zeros((L,), jnp.float32)

        def zbody(i, _):
            sl = pl.ds(i * L, L)
            se[sl] = zero
            ss[sl] = zero
            pe[sl] = zero
            ps[sl] = zero
            return 0

        lax.fori_loop(0, ACC // L, zbody, 0)

        lanes = lax.iota(jnp.int32, L)
        perm_next = jnp.minimum(lanes + 1, L - 1)
        perm_prev = jnp.maximum(lanes - 1, 0)
        lane_last = jnp.full((L,), L - 1, jnp.int32)

        def vbody(j, carry):
            carry_c, prev_d = carry
            sl = pl.ds(j * L, L)
            d = idx_v[sl]
            v = val_v[sl]
            d_next = jnp.where(
                lanes == L - 1,
                -1,
                jnp.take_along_axis(d, perm_next, axis=0, mode="promise_in_bounds"),
            )
            d_prev = jnp.where(
                lanes == 0,
                prev_d,
                jnp.take_along_axis(d, perm_prev, axis=0, mode="promise_in_bounds"),
            )
            end_m = d != d_next
            start_m = d != d_prev
            c = plsc.cumsum(v) + carry_c
            excl = c - v
            pos = (base + j * L + lanes).astype(jnp.float32)
            plsc.store_scatter(se, [d], c, mask=end_m)
            plsc.store_scatter(ss, [d], excl, mask=start_m)
            plsc.store_scatter(pe, [d], pos + 1.0, mask=end_m)
            plsc.store_scatter(ps, [d], pos, mask=start_m)
            new_c = jnp.take_along_axis(c, lane_last, axis=0, mode="promise_in_bounds")
            new_d = jnp.take_along_axis(d, lane_last, axis=0, mode="promise_in_bounds")
            return (new_c, new_d)

        nv = jnp.where(last, TAIL_NV, NV)
        lax.fori_loop(0, nv, vbody, (zero, jnp.full((L,), -1, jnp.int32)))

        def dbody(i, _):
            sl = pl.ds(i * L, L)
            se[sl] = se[sl] - ss[sl]
            pe[sl] = pe[sl] - ps[sl]
            return 0

        lax.fori_loop(0, N_MOLS // L, dbody, 0)

        pltpu.sync_copy(se.at[pl.ds(0, N_MOLS)], psum_hbm.at[wid])
        pltpu.sync_copy(pe.at[pl.ds(0, N_MOLS)], pcnt_hbm.at[wid])

    return body(idx, val)


def _pool_combine(psum, pcnt):
    @functools.partial(
        pl.kernel,
        out_type=jax.ShapeDtypeStruct((N_MOLS,), jnp.float32),
        mesh=_sc_mesh(),
        compiler_params=pltpu.CompilerParams(needs_layout_passes=False),
        scratch_types=[
            pltpu.VMEM((NW, MB), jnp.float32),
            pltpu.VMEM((NW, MB), jnp.float32),
            pltpu.VMEM((MB,), jnp.float32),
        ],
    )
    def body(psum_hbm, pcnt_hbm, out_hbm, s_v, c_v, o_v):
        wid = lax.axis_index("s") * 2 + lax.axis_index("c")
        col = wid * MB
        pltpu.sync_copy(psum_hbm.at[:, pl.ds(col, MB)], s_v)
        pltpu.sync_copy(pcnt_hbm.at[:, pl.ds(col, MB)], c_v)

        zero = jnp.zeros((L,), jnp.float32)

        def gbody(g, _):
            sl = pl.ds(g * L, L)

            def rbody(r, acc):
                sa, ca = acc
                return (sa + s_v[r, sl], ca + c_v[r, sl])

            s, c = lax.fori_loop(0, NW, rbody, (zero, zero))
            o_v[sl] = s / jnp.maximum(c, 1.0)
            return 0

        lax.fori_loop(0, MB // L, gbody, 0)
        pltpu.sync_copy(o_v, out_hbm.at[pl.ds(col, MB)])

    return body(psum, pcnt)


def kernel(vi, atom_mol_batch, W1, b1, W2, b2):
    atom_prop = _mlp(vi, W1, b1, W2, b2)
    psum, pcnt = _pool_partial(atom_mol_batch.astype(jnp.int32), atom_prop)
    mol_prop = _pool_combine(psum, pcnt)
    return mol_prop[:, None]
